# baseline (speedup 1.0000x reference)
import functools

import jax
import jax.numpy as jnp
from jax import lax
from jax.experimental import pallas as pl
from jax.experimental.pallas import tpu as pltpu
from jax.experimental.pallas import tpu_sc as plsc

NUM_EMB = 1000000
NUM_TOKENS = 16384
EMBED_DIM = 32
NUM_TAGS = 32

_info = plsc.get_sparse_core_info()
_NC, _NS = _info.num_cores, _info.num_subcores
_NW = _NC * _NS
_BPW = NUM_TOKENS // _NW

_TNP = 2048
_VB = 4 * _TNP
_NBLK = (NUM_EMB + _VB - 1) // _VB
_PROWS = _NBLK * _TNP


def _ts_body(w_ref, b_ref, tT_ref, o_ref):
  for g in range(4):
    o_ref[:, 32 * g:32 * (g + 1)] = (
        lax.dot_general(tT_ref[:, _TNP * g:_TNP * (g + 1)], w_ref[...],
                        (((0,), (1,)), ((), ())),
                        preferred_element_type=jnp.float32)
        + b_ref[...])


def _score_table(W, b2d, tableT):
  return pl.pallas_call(
      _ts_body,
      grid=(_NBLK,),
      in_specs=[
          pl.BlockSpec((NUM_TAGS, EMBED_DIM), lambda i: (0, 0)),
          pl.BlockSpec((1, NUM_TAGS), lambda i: (0, 0)),
          pl.BlockSpec((EMBED_DIM, _VB), lambda i: (0, i)),
      ],
      out_specs=pl.BlockSpec((_TNP, 128), lambda i: (i, 0)),
      out_shape=jax.ShapeDtypeStruct((_PROWS, 128), jnp.float32),
  )(W, b2d, tableT)


def _make_gather():
  mesh = plsc.VectorSubcoreMesh(core_axis_name="c", subcore_axis_name="s")

  @functools.partial(
      pl.kernel,
      mesh=mesh,
      compiler_params=pltpu.CompilerParams(use_tc_tiling_on_sc=False),
      out_type=jax.ShapeDtypeStruct((NUM_TOKENS * EMBED_DIM,), jnp.float32),
      scratch_types=[
          pltpu.VMEM((_BPW,), jnp.int32),
          pltpu.VMEM((_BPW,), jnp.int32),
          pltpu.VMEM((_BPW * 32,), jnp.float32),
          pltpu.SemaphoreType.DMA,
      ],
  )
  def gather_k(idx_hbm, pk_hbm, out_hbm, idx_v, off_v, outb_v, sem):
    wid = lax.axis_index("s") * _NC + lax.axis_index("c")
    base = wid * _BPW
    pltpu.sync_copy(idx_hbm.at[pl.ds(base, _BPW)], idx_v)

    def amath(g, carry):
      r = idx_v[pl.ds(g * 16, 16)]
      prow = jnp.bitwise_or(
          jnp.left_shift(jnp.right_shift(r, 13), 11),
          jnp.bitwise_and(r, 2047))
      loff = 32 * jnp.bitwise_and(jnp.right_shift(r, 11), 3)
      idx_v[pl.ds(g * 16, 16)] = prow
      off_v[pl.ds(g * 16, 16)] = loff
      return carry

    lax.fori_loop(0, _BPW // 16, amath, 0)

    def fire(g, carry):
      pv = idx_v[pl.ds(g * 16, 16)]
      ov = off_v[pl.ds(g * 16, 16)]
      for j in range(16):
        t = g * 16 + j
        p = lax.squeeze(lax.slice(pv, (j,), (j + 1,)), (0,))
        o = pl.multiple_of(
            lax.squeeze(lax.slice(ov, (j,), (j + 1,)), (0,)), EMBED_DIM)
        pltpu.async_copy(
            pk_hbm.at[p, pl.ds(o, EMBED_DIM)],
            outb_v.at[pl.ds(t * EMBED_DIM, EMBED_DIM)], sem)
      return carry

    lax.fori_loop(0, _BPW // 16, fire, 0)

    def drain(g, carry):
      for j in range(16):
        t = g * 16 + j
        pltpu.make_async_copy(
            pk_hbm.at[0, pl.ds(0, EMBED_DIM)],
            outb_v.at[pl.ds(t * EMBED_DIM, EMBED_DIM)], sem).wait()
      return carry

    lax.fori_loop(0, _BPW // 16, drain, 0)
    pltpu.sync_copy(
        outb_v, out_hbm.at[pl.ds(base * EMBED_DIM, _BPW * EMBED_DIM)])

  return gather_k


_gather = _make_gather()


def kernel(sent, emb_table, W, b):
  tableT = jnp.swapaxes(emb_table, 0, 1)
  scores_pk = _score_table(W, b.reshape(1, NUM_TAGS), tableT)
  flat = _gather(sent, scores_pk)
  return flat.reshape(NUM_TOKENS, NUM_TAGS)
